# baseline (speedup 1.0000x reference)
import functools

import jax
import jax.numpy as jnp
from jax import lax
from jax.experimental import pallas as pl
from jax.experimental.pallas import tpu as pltpu
from jax.experimental.pallas import tpu_sc as plsc

N = 10000
E = 320000
D = 128
B = 64

NC = 2
NS = 16
NW = NC * NS

NP = 10240
EP = 327680
ET = EP // NW
CH = 128
NCHUNK = ET // CH
RPT = NP // NS

_mesh = plsc.VectorSubcoreMesh(core_axis_name="c", subcore_axis_name="s")


@functools.partial(
    pl.kernel,
    out_type=jax.ShapeDtypeStruct((NC, NP), jnp.float32),
    mesh=_mesh,
    scratch_types=[
        pltpu.VMEM_SHARED((NP,), jnp.float32),
        pltpu.VMEM((CH,), jnp.int32),
        pltpu.VMEM((CH,), jnp.float32),
    ],
)
def _sc_degree(dst_hbm, zeros1_hbm, deg_hbm, acc_sh, idx_v, ones_v):
    cid = lax.axis_index("c")
    tid = lax.axis_index("s")
    wid = tid * NC + cid
    ebase = wid * ET

    pltpu.sync_copy(zeros1_hbm.at[pl.ds(tid * RPT, RPT)],
                    acc_sh.at[pl.ds(tid * RPT, RPT)])
    for i in range(CH // 16):
        ones_v[pl.ds(i * 16, 16)] = jnp.ones((16,), jnp.float32)
    plsc.subcore_barrier()

    def body(c, carry):
        pltpu.sync_copy(dst_hbm.at[pl.ds(ebase + c * CH, CH)], idx_v)
        pltpu.sync_copy(ones_v, acc_sh.at[idx_v], add=True)
        return carry

    lax.fori_loop(0, NCHUNK, body, 0)
    plsc.subcore_barrier()
    pltpu.sync_copy(acc_sh.at[pl.ds(tid * RPT, RPT)],
                    deg_hbm.at[cid, pl.ds(tid * RPT, RPT)])


@functools.partial(
    pl.kernel,
    out_type=jax.ShapeDtypeStruct((NC, NP, D), jnp.float32),
    mesh=_mesh,
    scratch_types=[
        pltpu.VMEM_SHARED((NP, D), jnp.float32),
        pltpu.VMEM((CH,), jnp.int32), pltpu.VMEM((CH,), jnp.int32),
        pltpu.VMEM((CH,), jnp.int32), pltpu.VMEM((CH,), jnp.int32),
        pltpu.VMEM((CH, D), jnp.float32), pltpu.VMEM((CH, D), jnp.float32),
        pltpu.SemaphoreType.DMA, pltpu.SemaphoreType.DMA,
        pltpu.SemaphoreType.DMA, pltpu.SemaphoreType.DMA,
    ],
)
def _sc_aggregate(hs2x_hbm, src_hbm, dst_hbm, zeros2_hbm, agg_hbm,
                  acc_sh, is0, is1, id0, id1, r0, r1,
                  si0, si1, sg0, sg1):
    cid = lax.axis_index("c")
    tid = lax.axis_index("s")
    wid = tid * NC + cid
    ebase = wid * ET
    tab_off = cid * NP
    IS, ID, RW = (is0, is1), (id0, id1), (r0, r1)
    SI, SG = (si0, si1), (sg0, sg1)

    pltpu.sync_copy(zeros2_hbm.at[pl.ds(tid * RPT, RPT)],
                    acc_sh.at[pl.ds(tid * RPT, RPT)])

    def start_idx(c, b):
        off = ebase + c * CH
        pltpu.async_copy(src_hbm.at[pl.ds(off, CH)], IS[b], SI[b])
        pltpu.async_copy(dst_hbm.at[pl.ds(off, CH)], ID[b], SI[b])

    def wait_idx(c, b):
        off = ebase + c * CH
        pltpu.make_async_copy(src_hbm.at[pl.ds(off, CH)], IS[b], SI[b]).wait()
        pltpu.make_async_copy(dst_hbm.at[pl.ds(off, CH)], ID[b], SI[b]).wait()
        for k in range(CH // 16):
            sl = pl.ds(k * 16, 16)
            IS[b][sl] = IS[b][sl] + tab_off

    def start_gather(b):
        pltpu.async_copy(hs2x_hbm.at[IS[b]], RW[b], SG[b])

    def wait_gather(b):
        pltpu.make_async_copy(hs2x_hbm.at[IS[b]], RW[b], SG[b]).wait()

    def scatter(b):
        pltpu.sync_copy(RW[b], acc_sh.at[ID[b]], add=True)

    plsc.subcore_barrier()

    start_idx(0, 0)
    wait_idx(0, 0)
    start_gather(0)
    start_idx(1, 1)

    def body(i, carry):
        c0 = 2 * i
        wait_gather(0)
        start_idx(c0 + 2, 0)
        wait_idx(c0 + 1, 1)
        start_gather(1)
        scatter(0)
        wait_gather(1)
        start_idx(c0 + 3, 1)
        wait_idx(c0 + 2, 0)
        start_gather(0)
        scatter(1)
        return carry

    lax.fori_loop(0, NCHUNK // 2 - 1, body, 0)
    wait_gather(0)
    wait_idx(NCHUNK - 1, 1)
    start_gather(1)
    scatter(0)
    wait_gather(1)
    scatter(1)

    plsc.subcore_barrier()
    pltpu.sync_copy(acc_sh.at[pl.ds(tid * RPT, RPT)],
                    agg_hbm.at[cid, pl.ds(tid * RPT, RPT)])


BLK = 512
NBLK = NP // BLK


def _tc_prescale_body(x_ref, w_ref, deg_ref, hs_ref, dinv_ref):
    deg = deg_ref[...]
    d = deg[:, 0:1] + deg[:, 1:2] + 1.0
    dinv = lax.rsqrt(jnp.maximum(d, 1.0))
    h = jnp.dot(x_ref[...], w_ref[...], preferred_element_type=jnp.float32)
    hs_ref[...] = (dinv * h)[None]
    dinv_ref[...] = dinv


def _tc_prescale(x_p, w1, deg_t):
    return pl.pallas_call(
        _tc_prescale_body,
        grid=(2, NBLK),
        in_specs=[
            pl.BlockSpec((BLK, D), lambda j, i: (i, 0)),
            pl.BlockSpec((D, D), lambda j, i: (0, 0)),
            pl.BlockSpec((BLK, 2), lambda j, i: (i, 0)),
        ],
        out_specs=[
            pl.BlockSpec((1, BLK, D), lambda j, i: (j, i, 0)),
            pl.BlockSpec((BLK, 1), lambda j, i: (i, 0)),
        ],
        out_shape=[
            jax.ShapeDtypeStruct((2, NP, D), jnp.float32),
            jax.ShapeDtypeStruct((NP, 1), jnp.float32),
        ],
    )(x_p, w1, deg_t)


def _tc_mid_body(agg_ref, hs1_ref, dinv_ref, b1_ref, w2_ref, hs2_ref):
    s = agg_ref[0] + agg_ref[1] + hs1_ref[0]
    dinv = dinv_ref[...]
    out1 = jnp.maximum(dinv * s + b1_ref[...], 0.0)
    h2 = jnp.dot(out1, w2_ref[...], preferred_element_type=jnp.float32)
    hs2_ref[...] = (dinv * h2)[None]


def _tc_mid(agg1, hs1, dinv, b1, w2):
    return pl.pallas_call(
        _tc_mid_body,
        grid=(2, NBLK),
        in_specs=[
            pl.BlockSpec((NC, BLK, D), lambda j, i: (0, i, 0)),
            pl.BlockSpec((1, BLK, D), lambda j, i: (0, i, 0)),
            pl.BlockSpec((BLK, 1), lambda j, i: (i, 0)),
            pl.BlockSpec((1, D), lambda j, i: (0, 0)),
            pl.BlockSpec((D, D), lambda j, i: (0, 0)),
        ],
        out_specs=pl.BlockSpec((1, BLK, D), lambda j, i: (j, i, 0)),
        out_shape=jax.ShapeDtypeStruct((2, NP, D), jnp.float32),
    )(agg1, hs1, dinv, b1, w2)


def _tc_final_body(agg_ref, hs2_ref, dinv_ref, b2_ref, batch_ref, out_ref,
                   pool_acc, cnt_acc):
    i = pl.program_id(0)

    @pl.when(i == 0)
    def _():
        pool_acc[...] = jnp.zeros_like(pool_acc)
        cnt_acc[...] = jnp.zeros_like(cnt_acc)

    s = agg_ref[0] + agg_ref[1] + hs2_ref[0]
    out2 = jnp.maximum(dinv_ref[...] * s + b2_ref[...], 0.0)
    iota = lax.broadcasted_iota(jnp.int32, (BLK, B), 1)
    cmp = (batch_ref[...] == iota).astype(jnp.float32)
    dn = (((0,), (0,)), ((), ()))
    pool_acc[...] += lax.dot_general(cmp, out2, dn,
                                     preferred_element_type=jnp.float32)
    cnt_acc[...] += lax.dot_general(cmp, jnp.ones((BLK, D), jnp.float32), dn,
                                    preferred_element_type=jnp.float32)

    @pl.when(i == NBLK - 1)
    def _():
        out_ref[...] = pool_acc[...] / jnp.maximum(cnt_acc[...], 1.0)


def _tc_final(agg2, hs2, dinv, b2, batch_p):
    return pl.pallas_call(
        _tc_final_body,
        grid=(NBLK,),
        in_specs=[
            pl.BlockSpec((NC, BLK, D), lambda i: (0, i, 0)),
            pl.BlockSpec((1, BLK, D), lambda i: (0, i, 0)),
            pl.BlockSpec((BLK, 1), lambda i: (i, 0)),
            pl.BlockSpec((1, D), lambda i: (0, 0)),
            pl.BlockSpec((BLK, 1), lambda i: (i, 0)),
        ],
        out_specs=pl.BlockSpec((B, D), lambda i: (0, 0)),
        out_shape=jax.ShapeDtypeStruct((B, D), jnp.float32),
        scratch_shapes=[
            pltpu.VMEM((B, D), jnp.float32),
            pltpu.VMEM((B, D), jnp.float32),
        ],
    )(agg2, hs2, dinv, b2, batch_p)


@jax.jit
def kernel(x, edge_index, batch, W1, b1, W2, b2):
    i32 = jnp.int32
    f32 = jnp.float32
    src = jnp.concatenate(
        [edge_index[0].astype(i32), jnp.zeros((EP - E,), i32)])
    dst = jnp.concatenate(
        [edge_index[1].astype(i32), jnp.full((EP - E,), NP - 1, i32)])
    batch_p = jnp.concatenate(
        [batch.astype(i32), jnp.full((NP - N,), B, i32)]).reshape(NP, 1)
    x_p = jnp.pad(x.astype(f32), ((0, NP - N), (0, 0)))
    zeros1 = jnp.zeros((NP,), f32)
    zeros2 = jnp.zeros((NP, D), f32)
    b1r = b1.astype(f32).reshape(1, D)
    b2r = b2.astype(f32).reshape(1, D)

    deg = _sc_degree(dst, zeros1)
    deg_t = deg.T
    hs1x, dinv = _tc_prescale(x_p, W1.astype(f32), deg_t)
    agg1 = _sc_aggregate(hs1x.reshape(2 * NP, D), src, dst, zeros2)
    hs2x = _tc_mid(agg1, hs1x, dinv, b1r, W2.astype(f32))
    agg2 = _sc_aggregate(hs2x.reshape(2 * NP, D), src, dst, zeros2)
    return _tc_final(agg2, hs2x, dinv, b2r, batch_p)


# speedup vs baseline: 1.3906x; 1.3906x over previous
import functools

import jax
import jax.numpy as jnp
from jax import lax
from jax.experimental import pallas as pl
from jax.experimental.pallas import tpu as pltpu
from jax.experimental.pallas import tpu_sc as plsc

N = 10000
E = 320000
D = 128
B = 64

NC = 2
NS = 16
NW = NC * NS

NP = 10240
EP = 327680
ET = EP // NW
CH = 128
NCHUNK = ET // CH
RPT = NP // NS

T0 = 120
T1 = 40

_mesh = plsc.VectorSubcoreMesh(core_axis_name="c", subcore_axis_name="s")


@functools.partial(
    pl.kernel,
    out_type=jax.ShapeDtypeStruct((NC, NP), jnp.float32),
    mesh=_mesh,
    scratch_types=[
        pltpu.VMEM_SHARED((NP,), jnp.float32),
        pltpu.VMEM((CH,), jnp.int32), pltpu.VMEM((CH,), jnp.int32),
        pltpu.VMEM((CH,), jnp.float32),
        pltpu.SemaphoreType.DMA, pltpu.SemaphoreType.DMA,
    ],
)
def _sc_degree(dst_hbm, zeros1_hbm, deg_hbm, acc_sh, ix0, ix1, ones_v,
               sd0, sd1):
    cid = lax.axis_index("c")
    tid = lax.axis_index("s")
    wid = tid * NC + cid
    ebase = wid * ET
    IX, SD = (ix0, ix1), (sd0, sd1)

    pltpu.sync_copy(zeros1_hbm.at[pl.ds(tid * RPT, RPT)],
                    acc_sh.at[pl.ds(tid * RPT, RPT)])
    for i in range(CH // 16):
        ones_v[pl.ds(i * 16, 16)] = jnp.ones((16,), jnp.float32)
    plsc.subcore_barrier()

    def start_idx(c, b):
        pltpu.async_copy(dst_hbm.at[pl.ds(ebase + c * CH, CH)], IX[b], SD[b])

    def wait_idx(c, b):
        pltpu.make_async_copy(dst_hbm.at[pl.ds(ebase + c * CH, CH)],
                              IX[b], SD[b]).wait()

    def scatter(b):
        pltpu.sync_copy(ones_v, acc_sh.at[IX[b]], add=True)

    start_idx(0, 0)
    start_idx(1, 1)

    def body(i, carry):
        c0 = 2 * i
        wait_idx(c0, 0)
        scatter(0)
        start_idx(c0 + 2, 0)
        wait_idx(c0 + 1, 1)
        scatter(1)
        start_idx(c0 + 3, 1)
        return carry

    lax.fori_loop(0, NCHUNK // 2 - 1, body, 0)
    wait_idx(NCHUNK - 2, 0)
    scatter(0)
    wait_idx(NCHUNK - 1, 1)
    scatter(1)
    plsc.subcore_barrier()
    pltpu.sync_copy(acc_sh.at[pl.ds(tid * RPT, RPT)],
                    deg_hbm.at[cid, pl.ds(tid * RPT, RPT)])


@functools.partial(
    pl.kernel,
    out_type=jax.ShapeDtypeStruct((NC, NP, D), jnp.float32),
    mesh=_mesh,
    scratch_types=[
        pltpu.VMEM_SHARED((NP, D), jnp.float32),
        pltpu.VMEM((CH,), jnp.int32), pltpu.VMEM((CH,), jnp.int32),
        pltpu.VMEM((CH,), jnp.int32), pltpu.VMEM((CH,), jnp.int32),
        pltpu.VMEM((CH, D), jnp.float32), pltpu.VMEM((CH, D), jnp.float32),
        pltpu.SemaphoreType.DMA, pltpu.SemaphoreType.DMA,
        pltpu.SemaphoreType.DMA, pltpu.SemaphoreType.DMA,
    ],
)
def _sc_aggregate(hs_hbm, src_hbm, dst_hbm, zeros2_hbm, agg_hbm,
                  acc_sh, is0, is1, id0, id1, r0, r1,
                  si0, si1, sg0, sg1):
    cid = lax.axis_index("c")
    tid = lax.axis_index("s")
    ebase = tid * (2 * ET) + cid * (T0 * CH)
    nch = jnp.where(cid == 0, T0, T1)
    IS, ID, RW = (is0, is1), (id0, id1), (r0, r1)
    SI, SG = (si0, si1), (sg0, sg1)

    pltpu.sync_copy(zeros2_hbm.at[pl.ds(tid * RPT, RPT)],
                    acc_sh.at[pl.ds(tid * RPT, RPT)])

    def start_idx(c, b):
        off = ebase + c * CH
        pltpu.async_copy(src_hbm.at[pl.ds(off, CH)], IS[b], SI[b])
        pltpu.async_copy(dst_hbm.at[pl.ds(off, CH)], ID[b], SI[b])

    def wait_idx(c, b):
        off = ebase + c * CH
        pltpu.make_async_copy(src_hbm.at[pl.ds(off, CH)], IS[b], SI[b]).wait()
        pltpu.make_async_copy(dst_hbm.at[pl.ds(off, CH)], ID[b], SI[b]).wait()

    def start_gather(b):
        pltpu.async_copy(hs_hbm.at[IS[b]], RW[b], SG[b])

    def wait_gather(b):
        pltpu.make_async_copy(hs_hbm.at[IS[b]], RW[b], SG[b]).wait()

    def scatter(b):
        pltpu.sync_copy(RW[b], acc_sh.at[ID[b]], add=True)

    plsc.subcore_barrier()

    start_idx(0, 0)
    wait_idx(0, 0)
    start_gather(0)
    start_idx(1, 1)

    def body(i, carry):
        c0 = 2 * i
        wait_gather(0)
        start_idx(c0 + 2, 0)
        wait_idx(c0 + 1, 1)
        start_gather(1)
        scatter(0)
        wait_gather(1)
        start_idx(c0 + 3, 1)
        wait_idx(c0 + 2, 0)
        start_gather(0)
        scatter(1)
        return carry

    lax.fori_loop(0, (nch - 2) // 2, body, 0)
    wait_gather(0)
    wait_idx(nch - 1, 1)
    start_gather(1)
    scatter(0)
    wait_gather(1)
    scatter(1)

    plsc.subcore_barrier()
    pltpu.sync_copy(acc_sh.at[pl.ds(tid * RPT, RPT)],
                    agg_hbm.at[cid, pl.ds(tid * RPT, RPT)])


BLK = 512
NBLK = NP // BLK


def _tc_prescale_body(x_ref, w_ref, deg_ref, hs_ref, dinv_ref):
    deg = deg_ref[...]
    d = deg[:, 0:1] + deg[:, 1:2] + 1.0
    dinv = lax.rsqrt(jnp.maximum(d, 1.0))
    h = jnp.dot(x_ref[...], w_ref[...], preferred_element_type=jnp.float32)
    hs_ref[...] = dinv * h
    dinv_ref[...] = dinv


def _tc_prescale(x_p, w1, deg_t):
    return pl.pallas_call(
        _tc_prescale_body,
        grid=(NBLK,),
        in_specs=[
            pl.BlockSpec((BLK, D), lambda i: (i, 0)),
            pl.BlockSpec((D, D), lambda i: (0, 0)),
            pl.BlockSpec((BLK, 2), lambda i: (i, 0)),
        ],
        out_specs=[
            pl.BlockSpec((BLK, D), lambda i: (i, 0)),
            pl.BlockSpec((BLK, 1), lambda i: (i, 0)),
        ],
        out_shape=[
            jax.ShapeDtypeStruct((NP, D), jnp.float32),
            jax.ShapeDtypeStruct((NP, 1), jnp.float32),
        ],
    )(x_p, w1, deg_t)


def _tc_mid_body(agg_ref, hs1_ref, dinv_ref, b1_ref, w2_ref, hs2_ref):
    s = agg_ref[0] + agg_ref[1] + hs1_ref[...]
    dinv = dinv_ref[...]
    out1 = jnp.maximum(dinv * s + b1_ref[...], 0.0)
    h2 = jnp.dot(out1, w2_ref[...], preferred_element_type=jnp.float32)
    hs2_ref[...] = dinv * h2


def _tc_mid(agg1, hs1, dinv, b1, w2):
    return pl.pallas_call(
        _tc_mid_body,
        grid=(NBLK,),
        in_specs=[
            pl.BlockSpec((NC, BLK, D), lambda i: (0, i, 0)),
            pl.BlockSpec((BLK, D), lambda i: (i, 0)),
            pl.BlockSpec((BLK, 1), lambda i: (i, 0)),
            pl.BlockSpec((1, D), lambda i: (0, 0)),
            pl.BlockSpec((D, D), lambda i: (0, 0)),
        ],
        out_specs=pl.BlockSpec((BLK, D), lambda i: (i, 0)),
        out_shape=jax.ShapeDtypeStruct((NP, D), jnp.float32),
    )(agg1, hs1, dinv, b1, w2)


def _tc_final_body(agg_ref, hs2_ref, dinv_ref, b2_ref, batch_ref, out_ref,
                   pool_acc, cnt_acc):
    i = pl.program_id(0)

    @pl.when(i == 0)
    def _():
        pool_acc[...] = jnp.zeros_like(pool_acc)
        cnt_acc[...] = jnp.zeros_like(cnt_acc)

    s = agg_ref[0] + agg_ref[1] + hs2_ref[...]
    out2 = jnp.maximum(dinv_ref[...] * s + b2_ref[...], 0.0)
    iota = lax.broadcasted_iota(jnp.int32, (BLK, B), 1)
    cmp = (batch_ref[...] == iota).astype(jnp.float32)
    dn = (((0,), (0,)), ((), ()))
    pool_acc[...] += lax.dot_general(cmp, out2, dn,
                                     preferred_element_type=jnp.float32)
    cnt_acc[...] += lax.dot_general(cmp, jnp.ones((BLK, D), jnp.float32), dn,
                                    preferred_element_type=jnp.float32)

    @pl.when(i == NBLK - 1)
    def _():
        out_ref[...] = pool_acc[...] / jnp.maximum(cnt_acc[...], 1.0)


def _tc_final(agg2, hs2, dinv, b2, batch_p):
    return pl.pallas_call(
        _tc_final_body,
        grid=(NBLK,),
        in_specs=[
            pl.BlockSpec((NC, BLK, D), lambda i: (0, i, 0)),
            pl.BlockSpec((BLK, D), lambda i: (i, 0)),
            pl.BlockSpec((BLK, 1), lambda i: (i, 0)),
            pl.BlockSpec((1, D), lambda i: (0, 0)),
            pl.BlockSpec((BLK, 1), lambda i: (i, 0)),
        ],
        out_specs=pl.BlockSpec((B, D), lambda i: (0, 0)),
        out_shape=jax.ShapeDtypeStruct((B, D), jnp.float32),
        scratch_shapes=[
            pltpu.VMEM((B, D), jnp.float32),
            pltpu.VMEM((B, D), jnp.float32),
        ],
    )(agg2, hs2, dinv, b2, batch_p)


@jax.jit
def kernel(x, edge_index, batch, W1, b1, W2, b2):
    i32 = jnp.int32
    f32 = jnp.float32
    src = jnp.concatenate(
        [edge_index[0].astype(i32), jnp.zeros((EP - E,), i32)])
    dst = jnp.concatenate(
        [edge_index[1].astype(i32), jnp.full((EP - E,), NP - 1, i32)])
    batch_p = jnp.concatenate(
        [batch.astype(i32), jnp.full((NP - N,), B, i32)]).reshape(NP, 1)
    x_p = jnp.pad(x.astype(f32), ((0, NP - N), (0, 0)))
    zeros1 = jnp.zeros((NP,), f32)
    zeros2 = jnp.zeros((NP, D), f32)
    b1r = b1.astype(f32).reshape(1, D)
    b2r = b2.astype(f32).reshape(1, D)

    deg = _sc_degree(dst, zeros1)
    deg_t = deg.T
    hs1, dinv = _tc_prescale(x_p, W1.astype(f32), deg_t)
    agg1 = _sc_aggregate(hs1, src, dst, zeros2)
    hs2 = _tc_mid(agg1, hs1, dinv, b1r, W2.astype(f32))
    agg2 = _sc_aggregate(hs2, src, dst, zeros2)
    return _tc_final(agg2, hs2, dinv, b2r, batch_p)
